# Initial kernel scaffold; baseline (speedup 1.0000x reference)
import functools

import jax
import jax.numpy as jnp
from jax import lax
from jax.experimental import pallas as pl
from jax.experimental.pallas import tpu as pltpu
from jax.experimental.pallas import tpu_sc as plsc

N = 10000
E = 320000
D = 128
DE = 4
G = 64

NC = 2
NS = 16
NW = NC * NS
RPW = 313
NPAD = NW * RPW
CH = 80
NB = 8
RB = N // NB
EB = 8000
RPT = N // NS
ZB = 125

_MESH = dict(core_axis_name="c", subcore_axis_name="s", num_cores=NC,
             num_subcores=NS)
_F32 = jnp.float32


def _t0_body(x_ref, w_ref, b_ref, a_ref):
    h = jnp.maximum(x_ref[...] * w_ref[...] + b_ref[...], 0.0)
    a_ref[0] = h
    a_ref[1] = h


_t0 = pl.pallas_call(
    _t0_body,
    grid=(NB,),
    in_specs=[
        pl.BlockSpec((RB, 1), lambda i: (i, 0)),
        pl.BlockSpec((1, D), lambda i: (0, 0)),
        pl.BlockSpec((1, D), lambda i: (0, 0)),
    ],
    out_specs=pl.BlockSpec((2, RB, D), lambda i: (0, i, 0)),
    out_shape=jax.ShapeDtypeStruct((2, N, D), _F32),
)


def _t1_body(a_ref, scc_ref, tr_ref, wm_ref, wr_ref, xm_ref, hr_ref):
    for b in range(2):
        a = a_ref[b]
        scc = scc_ref[b]
        tv = tr_ref[b]
        wm = wm_ref[b]
        wr = wr_ref[b]
        xm_ref[b] = (jnp.dot(a, wm * scc, preferred_element_type=_F32)
                     + jnp.dot(tv, wm, preferred_element_type=_F32))
        hr_ref[b] = (jnp.dot(a, wr * scc, preferred_element_type=_F32)
                     + jnp.dot(tv, wr, preferred_element_type=_F32))


_t1 = pl.pallas_call(
    _t1_body,
    grid=(NB,),
    in_specs=[
        pl.BlockSpec((2, RB, D), lambda i: (0, i, 0)),
        pl.BlockSpec((2, D, 1), lambda i: (0, 0, 0)),
        pl.BlockSpec((2, 1, D), lambda i: (0, 0, 0)),
        pl.BlockSpec((2, D, D), lambda i: (0, 0, 0)),
        pl.BlockSpec((2, D, D), lambda i: (0, 0, 0)),
    ],
    out_specs=[
        pl.BlockSpec((2, RB, D), lambda i: (0, i, 0)),
        pl.BlockSpec((2, RB, D), lambda i: (0, i, 0)),
    ],
    out_shape=[
        jax.ShapeDtypeStruct((2, N, D), _F32),
        jax.ShapeDtypeStruct((2, N, D), _F32),
    ],
)


def _make_post(step):
    aggrs = (('add', 'mean'), ('mean', 'max'), ('max', 'add'))[step]
    acts = (('relu', 'relu'), ('relu', 'tanh'), ('tanh', 'relu'))[step]
    has_max = 'max' in aggrs

    def body(*refs):
        if has_max:
            (hr_ref, sums_ref, mx_ref, segea_ref, wep_ref, bias_ref, g_ref,
             bb_ref, a_ref, scr_ref, trr_ref, ssum, ssq) = refs
        else:
            (hr_ref, sums_ref, segea_ref, wep_ref, bias_ref, g_ref,
             bb_ref, a_ref, scr_ref, trr_ref, ssum, ssq) = refs
        i = pl.program_id(0)
        seg4 = segea_ref[0, :, :DE] + segea_ref[1, :, :DE]
        cnt = segea_ref[0, :, DE:DE + 1] + segea_ref[1, :, DE:DE + 1]

        def do_branch(b):
            if aggrs[b] == 'max':
                agg = jnp.where(cnt > 0, mx_ref[...], 0.0)
            else:
                if step == 0:
                    s = sums_ref[b]
                else:
                    s = sums_ref[0] + sums_ref[1]
                agg = s + jnp.dot(seg4, wep_ref[b],
                                  preferred_element_type=_F32)
                if aggrs[b] == 'mean':
                    agg = agg / jnp.maximum(cnt, 1.0)
            t = hr_ref[b] + agg + bias_ref[b]
            av = jnp.maximum(t, 0.0) if acts[b] == 'relu' else jnp.tanh(t)
            a_ref[b] = av
            ps = jnp.sum(av, axis=0, keepdims=True)
            pq = jnp.sum(av * av, axis=0, keepdims=True)

            @pl.when(i == 0)
            def _():
                ssum[b] = ps
                ssq[b] = pq

            @pl.when(i > 0)
            def _():
                ssum[b] += ps
                ssq[b] += pq

        do_branch(0)
        do_branch(1)

        @pl.when(i == NB - 1)
        def _():
            for b in range(2):
                m = ssum[b] / N
                v = ssq[b] / N - m * m
                scf = g_ref[b] * lax.rsqrt(v + 1e-5)
                scr_ref[b] = scf
                trr_ref[b] = bb_ref[b] - m * scf

    in_specs = [pl.BlockSpec((2, RB, D), lambda i: (0, i, 0)),
                pl.BlockSpec((2, RB, D), lambda i: (0, i, 0))]
    if has_max:
        in_specs.append(pl.BlockSpec((RB, D), lambda i: (i, 0)))
    in_specs += [
        pl.BlockSpec((2, RB, 16), lambda i: (0, i, 0)),
        pl.BlockSpec((2, DE, D), lambda i: (0, 0, 0)),
        pl.BlockSpec((2, 1, D), lambda i: (0, 0, 0)),
        pl.BlockSpec((2, 1, D), lambda i: (0, 0, 0)),
        pl.BlockSpec((2, 1, D), lambda i: (0, 0, 0)),
    ]
    return pl.pallas_call(
        body,
        grid=(NB,),
        in_specs=in_specs,
        out_specs=[
            pl.BlockSpec((2, RB, D), lambda i: (0, i, 0)),
            pl.BlockSpec((2, 1, D), lambda i: (0, 0, 0)),
            pl.BlockSpec((2, 1, D), lambda i: (0, 0, 0)),
        ],
        out_shape=[
            jax.ShapeDtypeStruct((2, N, D), _F32),
            jax.ShapeDtypeStruct((2, 1, D), _F32),
            jax.ShapeDtypeStruct((2, 1, D), _F32),
        ],
        scratch_shapes=[pltpu.VMEM((2, 1, D), _F32),
                        pltpu.VMEM((2, 1, D), _F32)],
    )


_posts = [_make_post(j) for j in range(3)]


def _em_body(ea_ref, we_ref, out_ref):
    out_ref[...] = jnp.dot(ea_ref[...], we_ref[...],
                           preferred_element_type=_F32)


_em = pl.pallas_call(
    _em_body,
    grid=(E // EB,),
    in_specs=[
        pl.BlockSpec((EB, DE), lambda i: (i, 0)),
        pl.BlockSpec((DE, D), lambda i: (0, 0)),
    ],
    out_specs=pl.BlockSpec((EB, D), lambda i: (i, 0)),
    out_shape=jax.ShapeDtypeStruct((E, D), _F32),
)


def _head_body(a_ref, scr_ref, trr_ref, batch_ref, w1_ref, b1_ref, w12_ref,
               b12_ref, w2_ref, b2_ref, out_ref, pool, cntg):
    i = pl.program_id(0)
    bids = jnp.broadcast_to(batch_ref[0], (G, RB))
    p = (lax.broadcasted_iota(jnp.int32, (G, RB), 0) == bids).astype(_F32)

    @pl.when(i == 0)
    def _():
        pool[...] = jnp.zeros((2, G, D), _F32)
        cntg[...] = jnp.zeros((G, 1), _F32)

    pool[0] += jnp.dot(p, a_ref[0], preferred_element_type=_F32)
    pool[1] += jnp.dot(p, a_ref[1], preferred_element_type=_F32)
    cntg[...] += jnp.sum(p, axis=1, keepdims=True)

    @pl.when(i == NB - 1)
    def _():
        cnt = cntg[...]
        p1 = pool[0] * scr_ref[0] + cnt * trr_ref[0]
        p2 = pool[1] * scr_ref[1] + cnt * trr_ref[1]
        p2 = p2 / jnp.maximum(cnt, 1.0)
        o1 = jnp.maximum(jnp.dot(p1, w1_ref[...],
                                 preferred_element_type=_F32) + b1_ref[...],
                         0.0)
        o2 = jnp.maximum(jnp.dot(p2, w12_ref[...],
                                 preferred_element_type=_F32) + b12_ref[...],
                         0.0)
        out_ref[...] = jnp.dot(o1 + o2, w2_ref[...],
                               preferred_element_type=_F32) + b2_ref[...]


_head = pl.pallas_call(
    _head_body,
    grid=(NB,),
    in_specs=[
        pl.BlockSpec((2, RB, D), lambda i: (0, i, 0)),
        pl.BlockSpec((2, 1, D), lambda i: (0, 0, 0)),
        pl.BlockSpec((2, 1, D), lambda i: (0, 0, 0)),
        pl.BlockSpec((1, 1, RB), lambda i: (i, 0, 0)),
        pl.BlockSpec((D, D), lambda i: (0, 0)),
        pl.BlockSpec((1, D), lambda i: (0, 0)),
        pl.BlockSpec((D, D), lambda i: (0, 0)),
        pl.BlockSpec((1, D), lambda i: (0, 0)),
        pl.BlockSpec((D, 1), lambda i: (0, 0)),
        pl.BlockSpec((1, 1), lambda i: (0, 0)),
    ],
    out_specs=pl.BlockSpec((G, 1), lambda i: (0, 0)),
    out_shape=jax.ShapeDtypeStruct((G, 1), _F32),
    scratch_shapes=[pltpu.VMEM((2, G, D), _F32), pltpu.VMEM((G, 1), _F32)],
)


def _zero_acc(acc, zbuf, s, width):
    def zrow(i, _):
        for g in range(width // 16):
            zbuf[i, pl.ds(g * 16, 16)] = jnp.zeros((16,), _F32)
        return 0

    lax.fori_loop(0, ZB, zrow, 0)
    for j in range(RPT // ZB):
        pltpu.sync_copy(zbuf, acc.at[pl.ds(s * RPT + j * ZB, ZB)])


def _drain_acc(acc, out, c, s):
    for j in range(RPT // ZB):
        r = s * RPT + j * ZB
        pltpu.sync_copy(acc.at[pl.ds(r, ZB)], out.at[pl.ds(c * N + r, ZB)])


@functools.partial(
    pl.kernel,
    out_type=jax.ShapeDtypeStruct((2 * N, 16), _F32),
    mesh=plsc.VectorSubcoreMesh(**_MESH),
    scratch_types=[
        pltpu.VMEM_SHARED((N, 16), _F32),
        pltpu.VMEM((ZB, 16), _F32),
        pltpu.VMEM((CH,), jnp.int32),
        pltpu.VMEM((CH, 16), _F32),
    ],
)
def _sc_segea(ea16, dstv, out, acc, zbuf, didx, updv):
    c = lax.axis_index("c")
    s = lax.axis_index("s")
    _zero_acc(acc, zbuf, s, 16)
    plsc.subcore_barrier()
    base = (c * NS + s) * (E // NW)

    def chunk(k, _):
        off = pl.multiple_of(base + k * CH, 8)
        pltpu.sync_copy(dstv.at[pl.ds(off, CH)], didx)
        pltpu.sync_copy(ea16.at[pl.ds(off, CH)], updv)
        pltpu.sync_copy(updv, acc.at[didx], add=True)
        return 0

    lax.fori_loop(0, (E // NW) // CH, chunk, 0)
    plsc.subcore_barrier()
    _drain_acc(acc, out, c, s)


def _make_segsum(dual):
    edges_per = E // NS if dual else E // NW
    nchunks = edges_per // CH

    @functools.partial(
        pl.kernel,
        out_type=jax.ShapeDtypeStruct((2 * N, D), _F32),
        mesh=plsc.VectorSubcoreMesh(**_MESH),
        scratch_types=[
            pltpu.VMEM_SHARED((N, D), _F32),
            pltpu.VMEM((ZB, D), _F32),
            pltpu.VMEM((CH,), jnp.int32),
            pltpu.VMEM((CH,), jnp.int32),
            pltpu.VMEM((CH, D), _F32),
            pltpu.SemaphoreType.DMA,
        ],
    )
    def k(xm, srcv, dstv, out, acc, zbuf, sidx, didx, rows, sem):
        c = lax.axis_index("c")
        s = lax.axis_index("s")
        _zero_acc(acc, zbuf, s, D)
        plsc.subcore_barrier()
        if dual:
            base = c * E + s * edges_per
        else:
            base = (c * NS + s) * edges_per

        def chunk(k_, _):
            off = pl.multiple_of(base + k_ * CH, 8)
            pltpu.sync_copy(srcv.at[pl.ds(off, CH)], sidx)
            doff = off - c * E if dual else off
            pltpu.sync_copy(dstv.at[pl.ds(doff, CH)], didx)
            pltpu.async_copy(xm.at[sidx], rows, sem).wait()
            pltpu.sync_copy(rows, acc.at[didx], add=True)
            return 0

        lax.fori_loop(0, nchunks, chunk, 0)
        plsc.subcore_barrier()
        _drain_acc(acc, out, c, s)

    return k


_segsum_dual = _make_segsum(True)
_segsum_split = _make_segsum(False)


@functools.partial(
    pl.kernel,
    out_type=jax.ShapeDtypeStruct((NPAD, D), _F32),
    mesh=plsc.VectorSubcoreMesh(**_MESH),
    scratch_types=[
        pltpu.VMEM((RPW + 7, D), _F32),
        pltpu.VMEM((CH, D), _F32),
        pltpu.VMEM((CH, D), _F32),
        pltpu.VMEM((CH,), jnp.int32),
        pltpu.VMEM((CH,), jnp.int32),
        pltpu.VMEM((CH,), jnp.int32),
        pltpu.VMEM((48,), jnp.int32),
        pltpu.SemaphoreType.DMA,
        pltpu.SemaphoreType.DMA,
    ],
)
def _sc_segmax(xmv, emv, srcp, permp, dstp, bnd, out, acc, rows, erows,
               sidx, pidx, didx, bndv, sem, sem2):
    c = lax.axis_index("c")
    s = lax.axis_index("s")
    w = s * NC + c
    neg = jnp.full((16,), -3.0e38, _F32)

    def irow(i, _):
        for g in range(D // 16):
            acc[i, pl.ds(g * 16, 16)] = neg
        return 0

    lax.fori_loop(0, RPW + 7, irow, 0)
    pltpu.sync_copy(bnd, bndv)
    lane = lax.iota(jnp.int32, 16)

    def sget(i):
        q = i // 16
        r = i % 16
        ch = bndv[pl.ds(pl.multiple_of(q * 16, 16), 16)]
        return jnp.sum(jnp.where(lane == r, ch, 0), axis=0)

    lo = sget(w)
    hi = sget(w + 1)
    lo_al = (lo // 8) * 8
    nch = (hi - lo_al + (CH - 1)) // CH
    rowbase = w * RPW

    def chunk(kk, _):
        off = pl.multiple_of(lo_al + kk * CH, 8)
        pltpu.sync_copy(srcp.at[pl.ds(off, CH)], sidx)
        pltpu.sync_copy(permp.at[pl.ds(off, CH)], pidx)
        pltpu.sync_copy(dstp.at[pl.ds(off, CH)], didx)
        cp1 = pltpu.async_copy(xmv.at[sidx], rows, sem)
        cp2 = pltpu.async_copy(emv.at[pidx], erows, sem2)
        cp1.wait()
        cp2.wait()

        def subloop(sub, _):
            d16 = didx[pl.ds(pl.multiple_of(sub * 16, 16), 16)]
            e0 = off + sub * 16
            for e in range(16):
                eg = e0 + e
                valid = jnp.logical_and(eg >= lo, eg < hi)
                dsc = jnp.sum(jnp.where(lane == e, d16, 0), axis=0)
                dloc = jnp.where(valid, dsc - rowbase, RPW)
                rvec = jnp.zeros((16,), jnp.int32) + dloc
                er = sub * 16 + e
                for g in range(D // 16):
                    colv = lane + g * 16
                    av = plsc.load_gather(acc, [rvec, colv])
                    mv = (rows[er, pl.ds(g * 16, 16)]
                          + erows[er, pl.ds(g * 16, 16)])
                    plsc.store_scatter(acc, [rvec, colv],
                                       jnp.maximum(av, mv))
            return 0

        lax.fori_loop(0, CH // 16, subloop, 0)
        return 0

    lax.fori_loop(0, nch, chunk, 0)
    pltpu.sync_copy(acc.at[pl.ds(0, RPW)], out.at[pl.ds(rowbase, RPW)])


def kernel(x, edge_index, edge_attr, batch, lin0_W, lin0_b, convs_Wr,
           convs_Wm, convs_We, convs_b, norm1_g, norm1_b, norm2_g, norm2_b,
           lin1_W, lin1_b, lin12_W, lin12_b, lin2_W, lin2_b):
    src = edge_index[0]
    dst = edge_index[1]

    perm = jnp.argsort(dst).astype(jnp.int32)
    dst_s = jnp.take(dst, perm)
    src_s = jnp.take(src, perm)
    bnd = jnp.searchsorted(
        dst_s, jnp.arange(NW + 1, dtype=jnp.int32) * RPW).astype(jnp.int32)
    bnd = jnp.concatenate([bnd, jnp.full((48 - NW - 1,), E, jnp.int32)])
    zpad = jnp.zeros((CH,), jnp.int32)
    srcp = jnp.concatenate([src_s, zpad])
    permp = jnp.concatenate([perm, zpad])
    dstp = jnp.concatenate([dst_s, zpad])
    src2 = jnp.concatenate([src, src + N])
    ea16 = jnp.zeros((E, 16), _F32).at[:, :DE].set(edge_attr).at[:, DE].set(1.0)

    segea = _sc_segea(ea16, dst).reshape(2, N, 16)

    em_by_conv = {2: _em(edge_attr, convs_We[2]),
                  4: _em(edge_attr, convs_We[4])}

    a = _t0(x, lin0_W, lin0_b.reshape(1, D))
    scc = jnp.ones((2, D, 1), _F32)
    trow = jnp.zeros((2, 1, D), _F32)

    for j in range(3):
        wm = jnp.stack([convs_Wm[j], convs_Wm[3 + j]])
        wr = jnp.stack([convs_Wr[j], convs_Wr[3 + j]])
        wep = jnp.stack([convs_We[j], convs_We[3 + j]])
        bias = jnp.stack([convs_b[j], convs_b[3 + j]]).reshape(2, 1, D)
        gpair = jnp.stack([norm1_g, norm2_g]).reshape(2, 1, D)
        bpair = jnp.stack([norm1_b, norm2_b]).reshape(2, 1, D)
        xm, hr = _t1(a, scc, trow, wm, wr)
        if j == 0:
            sums = _segsum_dual(xm.reshape(2 * N, D), src2, dst)
            a, scr, trr = _posts[0](hr, sums.reshape(2, N, D), segea, wep,
                                    bias, gpair, bpair)
        elif j == 1:
            sums = _segsum_split(xm[0], src, dst)
            mx = _sc_segmax(xm[1], em_by_conv[4], srcp, permp, dstp, bnd)
            a, scr, trr = _posts[1](hr, sums.reshape(2, N, D), mx, segea,
                                    wep, bias, gpair, bpair)
        else:
            sums = _segsum_split(xm[1], src, dst)
            mx = _sc_segmax(xm[0], em_by_conv[2], srcp, permp, dstp, bnd)
            a, scr, trr = _posts[2](hr, sums.reshape(2, N, D), mx, segea,
                                    wep, bias, gpair, bpair)
        scc = jnp.transpose(scr, (0, 2, 1))
        trow = trr

    out = _head(a, scr, trr, batch.reshape(NB, 1, RB), lin1_W,
                lin1_b.reshape(1, D), lin12_W, lin12_b.reshape(1, D),
                lin2_W, lin2_b.reshape(1, 1))
    return out.reshape(-1)


# speedup vs baseline: 1.5766x; 1.5766x over previous
import functools

import jax
import jax.numpy as jnp
from jax import lax
from jax.experimental import pallas as pl
from jax.experimental.pallas import tpu as pltpu
from jax.experimental.pallas import tpu_sc as plsc

N = 10000
E = 320000
D = 128
DE = 4
G = 64

NC = 2
NS = 16
NW = NC * NS
RPW = 320
NPAD = NW * RPW
CH = 80
NB = 10
RB = N // NB
EB = 8000
TROWS = 624

_MESH = dict(core_axis_name="c", subcore_axis_name="s", num_cores=NC,
             num_subcores=NS)
_F32 = jnp.float32


def _t0_body(x_ref, w_ref, b_ref, a_ref):
    h = jnp.maximum(x_ref[...] * w_ref[...] + b_ref[...], 0.0)
    a_ref[0] = h
    a_ref[1] = h


_t0 = pl.pallas_call(
    _t0_body,
    grid=(NB,),
    in_specs=[
        pl.BlockSpec((RB, 1), lambda i: (i, 0)),
        pl.BlockSpec((1, D), lambda i: (0, 0)),
        pl.BlockSpec((1, D), lambda i: (0, 0)),
    ],
    out_specs=pl.BlockSpec((2, RB, D), lambda i: (0, i, 0)),
    out_shape=jax.ShapeDtypeStruct((2, N, D), _F32),
)


def _t1_body(a_ref, sc_ref, m_ref, bb_ref, wm_ref, wr_ref, xm_ref, hr_ref):
    for b in range(2):
        h = (a_ref[b] - m_ref[b]) * sc_ref[b] + bb_ref[b]
        xm_ref[b] = jnp.dot(h, wm_ref[b], preferred_element_type=_F32,
                            precision=lax.Precision.HIGHEST)
        hr_ref[b] = jnp.dot(h, wr_ref[b], preferred_element_type=_F32,
                            precision=lax.Precision.HIGHEST)


_t1 = pl.pallas_call(
    _t1_body,
    grid=(NB,),
    in_specs=[
        pl.BlockSpec((2, RB, D), lambda i: (0, i, 0)),
        pl.BlockSpec((2, 1, D), lambda i: (0, 0, 0)),
        pl.BlockSpec((2, 1, D), lambda i: (0, 0, 0)),
        pl.BlockSpec((2, 1, D), lambda i: (0, 0, 0)),
        pl.BlockSpec((2, D, D), lambda i: (0, 0, 0)),
        pl.BlockSpec((2, D, D), lambda i: (0, 0, 0)),
    ],
    out_specs=[
        pl.BlockSpec((2, RB, D), lambda i: (0, i, 0)),
        pl.BlockSpec((2, RB, D), lambda i: (0, i, 0)),
    ],
    out_shape=[
        jax.ShapeDtypeStruct((2, N, D), _F32),
        jax.ShapeDtypeStruct((2, N, D), _F32),
    ],
)


def _make_post(step):
    aggrs = (('add', 'mean'), ('mean', 'max'), ('max', 'add'))[step]
    acts = (('relu', 'relu'), ('relu', 'tanh'), ('tanh', 'relu'))[step]
    has_max = 'max' in aggrs

    def body(*refs):
        if has_max:
            (hr_ref, sums_ref, mx_ref, segea_ref, wep_ref, bias_ref, g_ref,
             bb_ref, a_ref, scr_ref, trr_ref, ssum, ssq, smm) = refs
        else:
            (hr_ref, sums_ref, segea_ref, wep_ref, bias_ref, g_ref,
             bb_ref, a_ref, scr_ref, trr_ref, ssum, ssq, smm) = refs
        i = pl.program_id(0)
        seg4 = segea_ref[0, :, :DE] + segea_ref[1, :, :DE]
        cnt = segea_ref[0, :, DE:DE + 1] + segea_ref[1, :, DE:DE + 1]

        def do_branch(b):
            if aggrs[b] == 'max':
                agg = jnp.where(cnt > 0, mx_ref[...], 0.0)
            else:
                if step == 0:
                    s = sums_ref[b]
                else:
                    s = sums_ref[0] + sums_ref[1]
                agg = s + jnp.dot(seg4, wep_ref[b],
                                  preferred_element_type=_F32,
                             precision=lax.Precision.HIGHEST)
                if aggrs[b] == 'mean':
                    agg = agg / jnp.maximum(cnt, 1.0)
            t = hr_ref[b] + agg + bias_ref[b]
            av = jnp.maximum(t, 0.0) if acts[b] == 'relu' else jnp.tanh(t)
            a_ref[b] = av
            mb = jnp.sum(av, axis=0, keepdims=True) / RB
            dv = av - mb
            vb = jnp.sum(dv * dv, axis=0, keepdims=True)

            @pl.when(i == 0)
            def _():
                ssum[b] = mb
                ssq[b] = vb
                smm[b] = mb * mb

            @pl.when(i > 0)
            def _():
                ssum[b] += mb
                ssq[b] += vb
                smm[b] += mb * mb

        do_branch(0)
        do_branch(1)

        @pl.when(i == NB - 1)
        def _():
            for b in range(2):
                m = ssum[b] / NB
                v = (ssq[b] + RB * (smm[b] - NB * m * m)) / N
                scf = g_ref[b] * lax.rsqrt(v + 1e-5)
                scr_ref[b] = scf
                trr_ref[b] = m

    in_specs = [pl.BlockSpec((2, RB, D), lambda i: (0, i, 0)),
                pl.BlockSpec((2, RB, D), lambda i: (0, i, 0))]
    if has_max:
        in_specs.append(pl.BlockSpec((RB, D), lambda i: (i, 0)))
    in_specs += [
        pl.BlockSpec((2, RB, D), lambda i: (0, i, 0)),
        pl.BlockSpec((2, DE, D), lambda i: (0, 0, 0)),
        pl.BlockSpec((2, 1, D), lambda i: (0, 0, 0)),
        pl.BlockSpec((2, 1, D), lambda i: (0, 0, 0)),
        pl.BlockSpec((2, 1, D), lambda i: (0, 0, 0)),
    ]
    return pl.pallas_call(
        body,
        grid=(NB,),
        in_specs=in_specs,
        out_specs=[
            pl.BlockSpec((2, RB, D), lambda i: (0, i, 0)),
            pl.BlockSpec((2, 1, D), lambda i: (0, 0, 0)),
            pl.BlockSpec((2, 1, D), lambda i: (0, 0, 0)),
        ],
        out_shape=[
            jax.ShapeDtypeStruct((2, N, D), _F32),
            jax.ShapeDtypeStruct((2, 1, D), _F32),
            jax.ShapeDtypeStruct((2, 1, D), _F32),
        ],
        scratch_shapes=[pltpu.VMEM((2, 1, D), _F32),
                        pltpu.VMEM((2, 1, D), _F32),
                        pltpu.VMEM((2, 1, D), _F32)],
    )


_posts = [_make_post(j) for j in range(3)]


def _em_body(ea_ref, we_ref, out_ref):
    out_ref[...] = jnp.dot(ea_ref[...], we_ref[...],
                           preferred_element_type=_F32,
                             precision=lax.Precision.HIGHEST)


_em = pl.pallas_call(
    _em_body,
    grid=(E // EB,),
    in_specs=[
        pl.BlockSpec((EB, DE), lambda i: (i, 0)),
        pl.BlockSpec((DE, D), lambda i: (0, 0)),
    ],
    out_specs=pl.BlockSpec((EB, D), lambda i: (i, 0)),
    out_shape=jax.ShapeDtypeStruct((E, D), _F32),
)


def _head_body(a_ref, scr_ref, mr_ref, bb_ref, batch_ref, w1_ref, b1_ref,
               w12_ref, b12_ref, w2_ref, b2_ref, out_ref, pool, cntg):
    i = pl.program_id(0)
    bids = jnp.broadcast_to(batch_ref[0], (G, RB))
    p = (lax.broadcasted_iota(jnp.int32, (G, RB), 0) == bids).astype(_F32)

    @pl.when(i == 0)
    def _():
        pool[...] = jnp.zeros((2, G, D), _F32)
        cntg[...] = jnp.zeros((G, 1), _F32)

    h0 = (a_ref[0] - mr_ref[0]) * scr_ref[0] + bb_ref[0]
    h1 = (a_ref[1] - mr_ref[1]) * scr_ref[1] + bb_ref[1]
    pool[0] += jnp.dot(p, h0, preferred_element_type=_F32,
                             precision=lax.Precision.HIGHEST)
    pool[1] += jnp.dot(p, h1, preferred_element_type=_F32,
                             precision=lax.Precision.HIGHEST)
    cntg[...] += jnp.sum(p, axis=1, keepdims=True)

    @pl.when(i == NB - 1)
    def _():
        cnt = cntg[...]
        p1 = pool[0]
        p2 = pool[1] / jnp.maximum(cnt, 1.0)
        o1 = jnp.maximum(jnp.dot(p1, w1_ref[...],
                                 preferred_element_type=_F32,
                             precision=lax.Precision.HIGHEST) + b1_ref[...],
                         0.0)
        o2 = jnp.maximum(jnp.dot(p2, w12_ref[...],
                                 preferred_element_type=_F32,
                             precision=lax.Precision.HIGHEST) + b12_ref[...],
                         0.0)
        out_ref[...] = jnp.dot(o1 + o2, w2_ref[...],
                               preferred_element_type=_F32,
                             precision=lax.Precision.HIGHEST) + b2_ref[...]


_head = pl.pallas_call(
    _head_body,
    grid=(NB,),
    in_specs=[
        pl.BlockSpec((2, RB, D), lambda i: (0, i, 0)),
        pl.BlockSpec((2, 1, D), lambda i: (0, 0, 0)),
        pl.BlockSpec((2, 1, D), lambda i: (0, 0, 0)),
        pl.BlockSpec((2, 1, D), lambda i: (0, 0, 0)),
        pl.BlockSpec((1, 1, RB), lambda i: (i, 0, 0)),
        pl.BlockSpec((D, D), lambda i: (0, 0)),
        pl.BlockSpec((1, D), lambda i: (0, 0)),
        pl.BlockSpec((D, D), lambda i: (0, 0)),
        pl.BlockSpec((1, D), lambda i: (0, 0)),
        pl.BlockSpec((D, 1), lambda i: (0, 0)),
        pl.BlockSpec((1, 1), lambda i: (0, 0)),
    ],
    out_specs=pl.BlockSpec((G, 1), lambda i: (0, 0)),
    out_shape=jax.ShapeDtypeStruct((G, 1), _F32),
    scratch_shapes=[pltpu.VMEM((2, G, D), _F32), pltpu.VMEM((G, 1), _F32)],
)


def _ntiles16(s):
    return jnp.where(s == NS - 1, TROWS // 16 + 1, TROWS // 16)


def _zero_acc(acc, zbuf, s, width):
    def zrow(i, _):
        for g in range(width // 16):
            zbuf[i, pl.ds(g * 16, 16)] = jnp.zeros((16,), _F32)
        return 0

    lax.fori_loop(0, 16, zrow, 0)

    def cp(i, _):
        off = pl.multiple_of(s * TROWS + i * 16, 16)
        pltpu.sync_copy(zbuf, acc.at[pl.ds(off, 16)])
        return 0

    lax.fori_loop(0, _ntiles16(s), cp, 0)


def _drain_acc(acc, out, c, s):
    def cp(i, _):
        r = pl.multiple_of(s * TROWS + i * 16, 16)
        pltpu.sync_copy(acc.at[pl.ds(r, 16)], out.at[pl.ds(c * N + r, 16)])
        return 0

    lax.fori_loop(0, _ntiles16(s), cp, 0)


def _make_segsum(dual):
    edges_per = E // NS if dual else E // NW
    nchunks = edges_per // CH

    @functools.partial(
        pl.kernel,
        out_type=jax.ShapeDtypeStruct((2 * N, D), _F32),
        mesh=plsc.VectorSubcoreMesh(**_MESH),
        scratch_types=[
            pltpu.VMEM_SHARED((N, D), _F32),
            pltpu.VMEM((16, D), _F32),
            pltpu.VMEM((CH,), jnp.int32),
            pltpu.VMEM((CH,), jnp.int32),
            pltpu.VMEM((CH, D), _F32),
            pltpu.SemaphoreType.DMA,
        ],
    )
    def k(xm, srcv, dstv, out, acc, zbuf, sidx, didx, rows, sem):
        c = lax.axis_index("c")
        s = lax.axis_index("s")
        _zero_acc(acc, zbuf, s, D)
        plsc.subcore_barrier()
        if dual:
            base = c * E + s * edges_per
        else:
            base = (c * NS + s) * edges_per

        def chunk(k_, _):
            off = pl.multiple_of(base + k_ * CH, 8)
            pltpu.sync_copy(srcv.at[pl.ds(off, CH)], sidx)
            doff = off - c * E if dual else off
            pltpu.sync_copy(dstv.at[pl.ds(doff, CH)], didx)
            pltpu.async_copy(xm.at[sidx], rows, sem).wait()
            pltpu.sync_copy(rows, acc.at[didx], add=True)
            return 0

        lax.fori_loop(0, nchunks, chunk, 0)
        plsc.subcore_barrier()
        _drain_acc(acc, out, c, s)

    return k


_segsum_dual = _make_segsum(True)
_segsum_split = _make_segsum(False)


@functools.partial(
    pl.kernel,
    out_type=jax.ShapeDtypeStruct((NPAD * D,), _F32),
    mesh=plsc.VectorSubcoreMesh(**_MESH),
    scratch_types=[
        pltpu.VMEM(((RPW + 8) * D,), _F32),
        pltpu.VMEM((CH, D), _F32),
        pltpu.VMEM((CH, D), _F32),
        pltpu.VMEM((CH,), jnp.int32),
        pltpu.VMEM((CH,), jnp.int32),
        pltpu.VMEM((CH,), jnp.int32),
        pltpu.VMEM((NW * 16,), jnp.int32),
        pltpu.SemaphoreType.DMA,
        pltpu.SemaphoreType.DMA,
    ],
)
def _sc_segmax(xmv, emv, srcp, permp, dstp, bnd, out, acc, rows, erows,
               sidx, pidx, didx, bndv, sem, sem2):
    c = lax.axis_index("c")
    s = lax.axis_index("s")
    w = s * NC + c
    neg = jnp.full((16,), -3.0e38, _F32)

    def irow(i, _):
        acc[pl.ds(pl.multiple_of(i * 16, 16), 16)] = neg
        return 0

    lax.fori_loop(0, (RPW + 8) * D // 16, irow, 0)
    pltpu.sync_copy(bnd, bndv)
    lane = lax.iota(jnp.int32, 16)

    brow = bndv[pl.ds(pl.multiple_of(w * 16, 16), 16)]
    lo = brow[0]
    hi = brow[1]
    lo_al = (lo // 8) * 8
    nch = (hi - lo_al + (CH - 1)) // CH
    rowbase = w * RPW

    def chunk(kk, _):
        off = pl.multiple_of(lo_al + kk * CH, 8)
        pltpu.sync_copy(srcp.at[pl.ds(off, CH)], sidx)
        pltpu.sync_copy(permp.at[pl.ds(off, CH)], pidx)
        pltpu.sync_copy(dstp.at[pl.ds(off, CH)], didx)
        cp1 = pltpu.async_copy(xmv.at[sidx], rows, sem)
        cp2 = pltpu.async_copy(emv.at[pidx], erows, sem2)
        cp1.wait()
        cp2.wait()

        def subloop(sub, _):
            d16 = didx[pl.ds(pl.multiple_of(sub * 16, 16), 16)]
            e0 = off + sub * 16
            for e in range(16):
                eg = e0 + e
                valid = jnp.logical_and(eg >= lo, eg < hi)
                er = sub * 16 + e
                dsc = d16[e]
                dloc = jnp.where(valid, dsc - rowbase, RPW)
                fbase = pl.multiple_of(dloc * D, 16)
                for g in range(D // 16):
                    sl = pl.ds(pl.multiple_of(fbase + g * 16, 16), 16)
                    mv = (rows[er, pl.ds(g * 16, 16)]
                          + erows[er, pl.ds(g * 16, 16)])
                    acc[sl] = jnp.maximum(acc[sl], mv)
            return 0

        lax.fori_loop(0, CH // 16, subloop, 0)
        return 0

    lax.fori_loop(0, nch, chunk, 0)
    pltpu.sync_copy(acc.at[pl.ds(0, RPW * D)],
                    out.at[pl.ds(rowbase * D, RPW * D)])


def kernel(x, edge_index, edge_attr, batch, lin0_W, lin0_b, convs_Wr,
           convs_Wm, convs_We, convs_b, norm1_g, norm1_b, norm2_g, norm2_b,
           lin1_W, lin1_b, lin12_W, lin12_b, lin2_W, lin2_b):
    src = edge_index[0]
    dst = edge_index[1]

    perm = jnp.argsort(dst).astype(jnp.int32)
    dst_s = jnp.take(dst, perm)
    src_s = jnp.take(src, perm)
    bnd = jnp.searchsorted(
        dst_s, jnp.arange(NW + 1, dtype=jnp.int32) * RPW).astype(jnp.int32)
    bnd = (jnp.zeros((NW, 16), jnp.int32).at[:, 0].set(bnd[:NW])
           .at[:, 1].set(bnd[1:NW + 1]).reshape(NW * 16))
    zpad = jnp.zeros((CH,), jnp.int32)
    srcp = jnp.concatenate([src_s, zpad])
    permp = jnp.concatenate([perm, zpad])
    dstp = jnp.concatenate([dst_s, zpad])
    src2 = jnp.concatenate([src, src + N])
    ea_pad = (jnp.zeros((E, D), _F32).at[:, :DE].set(edge_attr)
              .at[:, DE].set(1.0))
    eidx = jnp.arange(E, dtype=jnp.int32)

    segea = _segsum_split(ea_pad, eidx, dst).reshape(2, N, D)

    em_by_conv = {2: _em(edge_attr, convs_We[2]),
                  4: _em(edge_attr, convs_We[4])}

    a = _t0(x, lin0_W, lin0_b.reshape(1, D))
    scr = jnp.ones((2, 1, D), _F32)
    mrr = jnp.zeros((2, 1, D), _F32)
    bbp = jnp.zeros((2, 1, D), _F32)

    for j in range(3):
        wm = jnp.stack([convs_Wm[j], convs_Wm[3 + j]])
        wr = jnp.stack([convs_Wr[j], convs_Wr[3 + j]])
        wep = jnp.stack([convs_We[j], convs_We[3 + j]])
        bias = jnp.stack([convs_b[j], convs_b[3 + j]]).reshape(2, 1, D)
        gpair = jnp.stack([norm1_g, norm2_g]).reshape(2, 1, D)
        bpair = jnp.stack([norm1_b, norm2_b]).reshape(2, 1, D)
        xm, hr = _t1(a, scr, mrr, bbp, wm, wr)
        if j == 0:
            sums = _segsum_dual(xm.reshape(2 * N, D), src2, dst)
            a, scr, trr = _posts[0](hr, sums.reshape(2, N, D), segea, wep,
                                    bias, gpair, bpair)
        elif j == 1:
            sums = _segsum_split(xm[0], src, dst)
            mx = _sc_segmax(xm[1], em_by_conv[4], srcp, permp, dstp,
                            bnd).reshape(NPAD, D)
            a, scr, trr = _posts[1](hr, sums.reshape(2, N, D), mx, segea,
                                    wep, bias, gpair, bpair)
        else:
            sums = _segsum_split(xm[1], src, dst)
            mx = _sc_segmax(xm[0], em_by_conv[2], srcp, permp, dstp,
                            bnd).reshape(NPAD, D)
            a, scr, trr = _posts[2](hr, sums.reshape(2, N, D), mx, segea,
                                    wep, bias, gpair, bpair)
        mrr = trr
        bbp = bpair

    out = _head(a, scr, mrr, bbp, batch.reshape(NB, 1, RB), lin1_W,
                lin1_b.reshape(1, D), lin12_W, lin12_b.reshape(1, D),
                lin2_W, lin2_b.reshape(1, 1))
    return out.reshape(-1)
